# baseline (speedup 1.0000x reference)
import functools

import numpy as np
import jax
import jax.numpy as jnp
from jax import lax
from jax.experimental import pallas as pl
from jax.experimental.pallas import tpu as pltpu
from jax.experimental.pallas import tpu_sc as plsc

_MIN_GAP = 4
_MAX_GAP = 48
_REPEATED_SAMPLING = 4


def _temporal_indices(t: int):
    max_gap = min(_MAX_GAP, t - 1)
    gap = np.linspace(_MIN_GAP, max_gap, _REPEATED_SAMPLING).astype(np.int32)
    return [0] + [int(g) for g in gap]


def kernel(x):
    b, c, t, h, w = x.shape
    idx = _temporal_indices(t)
    k = len(idx)
    bc = b * c
    n_planes = bc * k
    n_pairs = n_planes // 2

    spp = 4
    rows = h // spp
    n_ch = 2 * spp
    nbuf = 4
    lag = 2

    info = plsc.get_sparse_core_info()
    nc, ns = info.num_cores, info.num_subcores

    xv = x.reshape(bc * t * h, w)

    mesh = plsc.VectorSubcoreMesh(core_axis_name="c", subcore_axis_name="s")
    scratch = [pltpu.VMEM((rows, w), x.dtype) for _ in range(nbuf)] + [
        pltpu.SemaphoreType.DMA for _ in range(2 * nbuf)
    ]

    @functools.partial(
        pl.kernel,
        mesh=mesh,
        out_type=jax.ShapeDtypeStruct((n_planes * h, w), x.dtype),
        scratch_types=scratch,
    )
    def sc_copy(x_hbm, o_hbm, *sc):
        bufs = sc[:nbuf]
        gsem = sc[nbuf:2 * nbuf]
        ssem = sc[2 * nbuf:3 * nbuf]
        wid = lax.axis_index("s") * nc + lax.axis_index("c")

        def src_plane(u):
            p = u // k
            r = u - p * k
            tsel = sum(v * (r == kk) for kk, v in enumerate(idx))
            return p * t + tsel

        @pl.when(wid < n_pairs)
        def _():
            gh = {}
            sh = {}

            def rowoffs(i):
                u = 2 * wid + i // spp
                s = (i % spp) * rows
                return src_plane(u) * h + s, u * h + s

            def start_gather(i):
                srow, _ = rowoffs(i)
                gh[i] = pltpu.async_copy(
                    x_hbm.at[pl.ds(srow, rows)], bufs[i % nbuf], gsem[i % nbuf])

            def start_scatter(i):
                _, drow = rowoffs(i)
                sh[i] = pltpu.async_copy(
                    bufs[i % nbuf], o_hbm.at[pl.ds(drow, rows)], ssem[i % nbuf])

            for i in range(n_ch + lag):
                if i < n_ch:
                    if i >= nbuf:
                        sh[i - nbuf].wait()
                    start_gather(i)
                j = i - lag
                if 0 <= j < n_ch:
                    gh[j].wait()
                    start_scatter(j)
            for j in range(n_ch - nbuf, n_ch):
                sh[j].wait()

    out = sc_copy(xv)
    return out.reshape(b, c, k, h, w)


# speedup vs baseline: 1.0219x; 1.0219x over previous
import functools

import numpy as np
import jax
import jax.numpy as jnp
from jax import lax
from jax.experimental import pallas as pl
from jax.experimental.pallas import tpu as pltpu
from jax.experimental.pallas import tpu_sc as plsc

_MIN_GAP = 4
_MAX_GAP = 48
_REPEATED_SAMPLING = 4


def _temporal_indices(t: int):
    max_gap = min(_MAX_GAP, t - 1)
    gap = np.linspace(_MIN_GAP, max_gap, _REPEATED_SAMPLING).astype(np.int32)
    return [0] + [int(g) for g in gap]


def kernel(x):
    b, c, t, h, w = x.shape
    idx = _temporal_indices(t)
    k = len(idx)
    bc = b * c
    n_planes = bc * k
    n_pairs = n_planes // 2

    info = plsc.get_sparse_core_info()
    nc, ns = info.num_cores, info.num_subcores

    xv = x.reshape(bc * t, h, w)

    mesh = plsc.VectorSubcoreMesh(core_axis_name="c", subcore_axis_name="s")
    scratch = [
        pltpu.VMEM((h, w), x.dtype),
        pltpu.VMEM((h, w), x.dtype),
        pltpu.SemaphoreType.DMA,
        pltpu.SemaphoreType.DMA,
        pltpu.SemaphoreType.DMA,
        pltpu.SemaphoreType.DMA,
    ]

    @functools.partial(
        pl.kernel,
        mesh=mesh,
        out_type=jax.ShapeDtypeStruct((n_planes, h, w), x.dtype),
        scratch_types=scratch,
    )
    def sc_copy(x_hbm, o_hbm, buf0, buf1, gs0, gs1, ss0, ss1):
        wid = lax.axis_index("s") * nc + lax.axis_index("c")

        def src_plane(u):
            p = u // k
            r = u - p * k
            tsel = sum(v * (r == kk) for kk, v in enumerate(idx))
            return p * t + tsel

        u0 = 2 * wid
        u1 = 2 * wid + 1

        @pl.when(wid < n_pairs)
        def _():
            g0 = pltpu.async_copy(x_hbm.at[src_plane(u0)], buf0, gs0)
            g1 = pltpu.async_copy(x_hbm.at[src_plane(u1)], buf1, gs1)
            g0.wait()
            s0 = pltpu.async_copy(buf0, o_hbm.at[u0], ss0)
            g1.wait()
            s1 = pltpu.async_copy(buf1, o_hbm.at[u1], ss1)
            s0.wait()
            s1.wait()

    out = sc_copy(xv)
    return out.reshape(b, c, k, h, w)
